# baseline (device time: 169326 ns/iter reference)
import jax
import jax.numpy as jnp
from jax import lax
from jax.experimental import pallas as pl
from jax.experimental.pallas import tpu as pltpu

CH = 128
LS = 6
S3 = (768, 640, 640)
OFF3 = (0, 768, 1408)

SZ = {rel: S3[rel % 3] for rel in range(12)}
NCK = {rel: SZ[rel] // CH for rel in range(12)}

Y_QUEUE = (1, 2, 0, 3)
YB_OFF = {1: 0, 2: 640, 0: 1280, 3: 2048}
YPOS = {1: 0, 2: 5, 0: 10, 3: 16}

R_QUEUE = (1, 10, 2, 0)
RPOS = {1: 0, 10: 5, 2: 10, 0: 15}
L_QUEUE = (1, 2, 5, 3)
LPOS = {1: 0, 2: 5, 5: 10, 3: 15}


ADD_ORDER = (
    (1, "y", 0), (10, "l", 0), (4, "r", 0),
    (2, "y", 640), (5, "r", 640),
    (0, "y", 1280), (7, "l", 640),
    (11, "l", 1280), (8, "r", 1280),
    (9, "l", 1920), (6, "r", 1920),
    (3, "y", 2048),
)


def kernel(x):
    m, n = x.shape

    def body(x_ref, out_ref, ybuf, linbuf, rinbuf, lbuf,
             in_sems, out_sems, ysend, yrecv, rsend, linrecv, lsend, rinrecv):
        my_x = lax.axis_index("x")
        my_y = lax.axis_index("y")
        my_z = lax.axis_index("z")
        zl = lax.rem(my_z, 2)
        zpz = my_z + 1 - 2 * zl
        xz = lax.rem(my_x + zl, 2)
        r_ring = 2 * zl + xz
        e = xz == 0

        partner = (my_x, 1 - my_y, my_z)
        xn = (1 - my_x, my_y, my_z)
        zn = (my_x, my_y, zpz)
        right_dev = (jnp.where(e, 1 - my_x, my_x), my_y,
                     jnp.where(e, my_z, zpz))
        left_dev = (jnp.where(e, my_x, 1 - my_x), my_y,
                    jnp.where(e, zpz, my_z))

        def off(rel):
            return 2048 * lax.rem(r_ring + rel // 3, 4) + OFF3[rel % 3]

        barrier_sem = pltpu.get_barrier_semaphore()
        for nbr in (partner, xn, zn):
            pl.semaphore_signal(
                barrier_sem, inc=1,
                device_id=nbr, device_id_type=pl.DeviceIdType.MESH,
            )
        pl.semaphore_wait(barrier_sem, 3)

        y_rd = {}
        y_rdmas = []
        p = 0
        for rel in Y_QUEUE:
            for c in range(NCK[rel]):
                rr = pltpu.make_async_remote_copy(
                    src_ref=x_ref.at[pl.ds(off(rel) + c * CH, CH)],
                    dst_ref=ybuf.at[pl.ds(YB_OFF[rel] + c * CH, CH)],
                    send_sem=ysend.at[p],
                    recv_sem=yrecv.at[p],
                    device_id=partner,
                    device_id_type=pl.DeviceIdType.MESH,
                )
                rr.start()
                y_rd[(rel, c)] = rr
                y_rdmas.append(rr)
                p += 1

        fwd_rdmas = []

        def rfwd(rel, c):
            if rel == 10:
                src = linbuf.at[pl.ds(0 + c * CH, CH)]
            else:
                src = ybuf.at[pl.ds(YB_OFF[rel] + c * CH, CH)]
            q = RPOS[rel] + c
            rr = pltpu.make_async_remote_copy(
                src_ref=src,
                dst_ref=linbuf.at[pl.ds(q * CH, CH)],
                send_sem=rsend.at[q],
                recv_sem=linrecv.at[q],
                device_id=right_dev,
                device_id_type=pl.DeviceIdType.MESH,
            )
            rr.start()
            fwd_rdmas.append(rr)

        def lfwd(rel, c):
            if rel == 5:
                src = rinbuf.at[pl.ds(640 + c * CH, CH)]
            else:
                src = ybuf.at[pl.ds(YB_OFF[rel] + c * CH, CH)]
            q = LPOS[rel] + c
            rr = pltpu.make_async_remote_copy(
                src_ref=src,
                dst_ref=rinbuf.at[pl.ds(q * CH, CH)],
                send_sem=lsend.at[q],
                recv_sem=rinrecv.at[q],
                device_id=left_dev,
                device_id_type=pl.DeviceIdType.MESH,
            )
            rr.start()
            fwd_rdmas.append(rr)

        def wait_lin(q):
            pltpu.make_async_remote_copy(
                src_ref=linbuf.at[pl.ds(q * CH, CH)],
                dst_ref=linbuf.at[pl.ds(q * CH, CH)],
                send_sem=linrecv.at[q], recv_sem=linrecv.at[q],
                device_id=left_dev, device_id_type=pl.DeviceIdType.MESH,
            ).wait_recv()

        def wait_rin(q):
            pltpu.make_async_remote_copy(
                src_ref=rinbuf.at[pl.ds(q * CH, CH)],
                dst_ref=rinbuf.at[pl.ds(q * CH, CH)],
                send_sem=rinrecv.at[q], recv_sem=rinrecv.at[q],
                device_id=right_dev, device_id_type=pl.DeviceIdType.MESH,
            ).wait_recv()

        bufs = {"y": ybuf, "l": linbuf, "r": rinbuf}
        cp_ins = {}
        out_cps = {}

        def start_load(aj):
            rel = ADD_ORDER[aj][0]
            c = pltpu.make_async_copy(
                x_ref.at[pl.ds(off(rel), SZ[rel])],
                lbuf.at[aj % LS, pl.ds(0, SZ[rel])],
                in_sems.at[aj % LS],
            )
            c.start()
            cp_ins[aj] = c

        def do_add(aj):
            rel, key, o = ADD_ORDER[aj]
            szr = SZ[rel]
            s = aj % LS
            cp_ins[aj].wait()
            lbuf[s, :szr] = lbuf[s, :szr] + bufs[key][o:o + szr]
            oc = pltpu.make_async_copy(
                lbuf.at[s, pl.ds(0, szr)],
                out_ref.at[pl.ds(off(rel), szr)],
                out_sems.at[s],
            )
            oc.start()
            out_cps[aj] = oc
            if aj + LS < 12:
                oc.wait()
                start_load(aj + LS)

        for aj in range(LS):
            start_load(aj)

        for c in range(5):
            y_rd[(1, c)].wait_recv()
            rfwd(1, c)
            lfwd(1, c)
        for c in range(5):
            wait_lin(c)
            rfwd(10, c)
        do_add(0)
        do_add(1)
        for c in range(5):
            wait_rin(c)
        do_add(2)
        for c in range(5):
            y_rd[(2, c)].wait_recv()
            rfwd(2, c)
            lfwd(2, c)
        for c in range(5):
            wait_rin(5 + c)
            lfwd(5, c)
        do_add(3)
        do_add(4)
        for c in range(6):
            y_rd[(0, c)].wait_recv()
            rfwd(0, c)
        do_add(5)
        for c in range(5):
            wait_lin(5 + c)
        do_add(6)
        cp_ins[11].wait()
        for c in range(3):
            y_rd[(3, c)].wait_recv()
            lfwd(3, c)
            lbuf[11 % LS, c * CH:(c + 1) * CH] = (
                lbuf[11 % LS, c * CH:(c + 1) * CH]
                + ybuf[2048 + c * CH:2048 + (c + 1) * CH])
        for c in range(5):
            wait_lin(10 + c)
        do_add(7)
        for c in range(5):
            wait_rin(10 + c)
        do_add(8)
        cp_ins[9].wait()
        cp_ins[10].wait()
        for c in range(3):
            wait_lin(15 + c)
            lbuf[9 % LS, c * CH:(c + 1) * CH] = (
                lbuf[9 % LS, c * CH:(c + 1) * CH]
                + linbuf[1920 + c * CH:1920 + (c + 1) * CH])
            wait_rin(15 + c)
            lbuf[10 % LS, c * CH:(c + 1) * CH] = (
                lbuf[10 % LS, c * CH:(c + 1) * CH]
                + rinbuf[1920 + c * CH:1920 + (c + 1) * CH])
        for c in range(3, 6):
            y_rd[(3, c)].wait_recv()
            lfwd(3, c)
            lbuf[11 % LS, c * CH:(c + 1) * CH] = (
                lbuf[11 % LS, c * CH:(c + 1) * CH]
                + ybuf[2048 + c * CH:2048 + (c + 1) * CH])
        for c in range(3, 6):
            wait_lin(15 + c)
            lbuf[9 % LS, c * CH:(c + 1) * CH] = (
                lbuf[9 % LS, c * CH:(c + 1) * CH]
                + linbuf[1920 + c * CH:1920 + (c + 1) * CH])
            wait_rin(15 + c)
            lbuf[10 % LS, c * CH:(c + 1) * CH] = (
                lbuf[10 % LS, c * CH:(c + 1) * CH]
                + rinbuf[1920 + c * CH:1920 + (c + 1) * CH])
        for aj in (9, 10, 11):
            rel = ADD_ORDER[aj][0]
            oc = pltpu.make_async_copy(
                lbuf.at[aj % LS, pl.ds(0, SZ[rel])],
                out_ref.at[pl.ds(off(rel), SZ[rel])],
                out_sems.at[aj % LS],
            )
            oc.start()
            out_cps[aj] = oc

        for aj in range(6, 12):
            out_cps[aj].wait()
        for rr in y_rdmas:
            rr.wait_send()
        for rr in fwd_rdmas:
            rr.wait_send()

    return pl.pallas_call(
        body,
        out_shape=jax.ShapeDtypeStruct((m, n), x.dtype),
        in_specs=[pl.BlockSpec(memory_space=pltpu.MemorySpace.HBM)],
        out_specs=pl.BlockSpec(memory_space=pltpu.MemorySpace.HBM),
        scratch_shapes=[
            pltpu.VMEM((2816, n), x.dtype),
            pltpu.VMEM((2688, n), x.dtype),
            pltpu.VMEM((2688, n), x.dtype),
            pltpu.VMEM((LS, 768, n), x.dtype),
            pltpu.SemaphoreType.DMA((LS,)),
            pltpu.SemaphoreType.DMA((LS,)),
            pltpu.SemaphoreType.DMA((22,)),
            pltpu.SemaphoreType.DMA((22,)),
            pltpu.SemaphoreType.DMA((21,)),
            pltpu.SemaphoreType.DMA((21,)),
            pltpu.SemaphoreType.DMA((21,)),
            pltpu.SemaphoreType.DMA((21,)),
        ],
        compiler_params=pltpu.CompilerParams(
            collective_id=0, vmem_limit_bytes=60 * 1024 * 1024,
        ),
    )(x)


# device time: 165920 ns/iter; 1.0205x vs baseline; 1.0205x over previous
import jax
import jax.numpy as jnp
from jax import lax
from jax.experimental import pallas as pl
from jax.experimental.pallas import tpu as pltpu

CH = 128
LS = 4
S3 = (768, 640, 640)
OFF3 = (0, 768, 1408)

SZ = {rel: S3[rel % 3] for rel in range(12)}
NCK = {rel: SZ[rel] // CH for rel in range(12)}

Y_QUEUE = (1, 2, 0, 3)
YB_OFF = {1: 0, 2: 640, 0: 1280, 3: 2048}
YPOS = {1: 0, 2: 5, 0: 10, 3: 16}

R_QUEUE = (1, 10, 2, 0)
RPOS = {1: 0, 10: 5, 2: 10, 0: 15}
L_QUEUE = (1, 2, 5, 3)
LPOS = {1: 0, 2: 5, 5: 10, 3: 15}


ADD_ORDER = (
    (1, "y", 0), (10, "l", 0), (4, "r", 0),
    (2, "y", 640), (5, "r", 640),
    (0, "y", 1280), (7, "l", 640),
    (11, "l", 1280), (8, "r", 1280),
    (9, "l", 1920), (6, "r", 1920),
    (3, "y", 2048),
)


def kernel(x):
    m, n = x.shape

    def body(x_ref, out_ref, ybuf, linbuf, rinbuf, lbuf,
             in_sems, out_sems, ysend, yrecv, rsend, linrecv, lsend, rinrecv):
        my_x = lax.axis_index("x")
        my_y = lax.axis_index("y")
        my_z = lax.axis_index("z")
        zl = lax.rem(my_z, 2)
        zpz = my_z + 1 - 2 * zl
        xz = lax.rem(my_x + zl, 2)
        r_ring = 2 * zl + xz
        e = xz == 0

        partner = (my_x, 1 - my_y, my_z)
        xn = (1 - my_x, my_y, my_z)
        zn = (my_x, my_y, zpz)
        right_dev = (jnp.where(e, 1 - my_x, my_x), my_y,
                     jnp.where(e, my_z, zpz))
        left_dev = (jnp.where(e, my_x, 1 - my_x), my_y,
                    jnp.where(e, zpz, my_z))

        def off(rel):
            return 2048 * lax.rem(r_ring + rel // 3, 4) + OFF3[rel % 3]

        barrier_sem = pltpu.get_barrier_semaphore()
        for nbr in (partner, xn, zn):
            pl.semaphore_signal(
                barrier_sem, inc=1,
                device_id=nbr, device_id_type=pl.DeviceIdType.MESH,
            )
        pl.semaphore_wait(barrier_sem, 3)

        y_rd = {}
        y_rdmas = []
        p = 0
        for rel in Y_QUEUE:
            for c in range(NCK[rel]):
                rr = pltpu.make_async_remote_copy(
                    src_ref=x_ref.at[pl.ds(off(rel) + c * CH, CH)],
                    dst_ref=ybuf.at[pl.ds(YB_OFF[rel] + c * CH, CH)],
                    send_sem=ysend.at[p],
                    recv_sem=yrecv.at[p],
                    device_id=partner,
                    device_id_type=pl.DeviceIdType.MESH,
                )
                rr.start()
                y_rd[(rel, c)] = rr
                y_rdmas.append(rr)
                p += 1

        fwd_rdmas = []

        def rfwd(rel, c):
            if rel == 10:
                src = linbuf.at[pl.ds(0 + c * CH, CH)]
            else:
                src = ybuf.at[pl.ds(YB_OFF[rel] + c * CH, CH)]
            q = RPOS[rel] + c
            rr = pltpu.make_async_remote_copy(
                src_ref=src,
                dst_ref=linbuf.at[pl.ds(q * CH, CH)],
                send_sem=rsend.at[q],
                recv_sem=linrecv.at[q],
                device_id=right_dev,
                device_id_type=pl.DeviceIdType.MESH,
            )
            rr.start()
            fwd_rdmas.append(rr)

        def lfwd(rel, c):
            if rel == 5:
                src = rinbuf.at[pl.ds(640 + c * CH, CH)]
            else:
                src = ybuf.at[pl.ds(YB_OFF[rel] + c * CH, CH)]
            q = LPOS[rel] + c
            rr = pltpu.make_async_remote_copy(
                src_ref=src,
                dst_ref=rinbuf.at[pl.ds(q * CH, CH)],
                send_sem=lsend.at[q],
                recv_sem=rinrecv.at[q],
                device_id=left_dev,
                device_id_type=pl.DeviceIdType.MESH,
            )
            rr.start()
            fwd_rdmas.append(rr)

        def wait_lin(q):
            pltpu.make_async_remote_copy(
                src_ref=linbuf.at[pl.ds(q * CH, CH)],
                dst_ref=linbuf.at[pl.ds(q * CH, CH)],
                send_sem=linrecv.at[q], recv_sem=linrecv.at[q],
                device_id=left_dev, device_id_type=pl.DeviceIdType.MESH,
            ).wait_recv()

        def wait_rin(q):
            pltpu.make_async_remote_copy(
                src_ref=rinbuf.at[pl.ds(q * CH, CH)],
                dst_ref=rinbuf.at[pl.ds(q * CH, CH)],
                send_sem=rinrecv.at[q], recv_sem=rinrecv.at[q],
                device_id=right_dev, device_id_type=pl.DeviceIdType.MESH,
            ).wait_recv()

        bufs = {"y": ybuf, "l": linbuf, "r": rinbuf}
        cp_ins = {}
        out_cps = {}

        def start_load(aj):
            rel = ADD_ORDER[aj][0]
            c = pltpu.make_async_copy(
                x_ref.at[pl.ds(off(rel), SZ[rel])],
                lbuf.at[aj % LS, pl.ds(0, SZ[rel])],
                in_sems.at[aj % LS],
            )
            c.start()
            cp_ins[aj] = c

        def do_add(aj):
            rel, key, o = ADD_ORDER[aj]
            szr = SZ[rel]
            s = aj % LS
            cp_ins[aj].wait()
            lbuf[s, :szr] = lbuf[s, :szr] + bufs[key][o:o + szr]
            oc = pltpu.make_async_copy(
                lbuf.at[s, pl.ds(0, szr)],
                out_ref.at[pl.ds(off(rel), szr)],
                out_sems.at[s],
            )
            oc.start()
            out_cps[aj] = oc
            if aj + LS < 12:
                oc.wait()
                start_load(aj + LS)

        for aj in range(LS):
            start_load(aj)

        for c in range(5):
            y_rd[(1, c)].wait_recv()
            rfwd(1, c)
            lfwd(1, c)
        for c in range(5):
            wait_lin(c)
            rfwd(10, c)
        do_add(0)
        do_add(1)
        for c in range(5):
            wait_rin(c)
        do_add(2)
        for c in range(5):
            y_rd[(2, c)].wait_recv()
            rfwd(2, c)
            lfwd(2, c)
        for c in range(5):
            wait_rin(5 + c)
            lfwd(5, c)
        do_add(3)
        do_add(4)
        for c in range(6):
            y_rd[(0, c)].wait_recv()
            rfwd(0, c)
        do_add(5)
        for c in range(5):
            wait_lin(5 + c)
        do_add(6)
        for c in range(3):
            y_rd[(3, c)].wait_recv()
            lfwd(3, c)
        for c in range(5):
            wait_lin(10 + c)
        do_add(7)
        for c in range(5):
            wait_rin(10 + c)
        do_add(8)
        for c in range(3, 6):
            y_rd[(3, c)].wait_recv()
            lfwd(3, c)
        for c in range(6):
            wait_lin(15 + c)
        do_add(9)
        for c in range(6):
            wait_rin(15 + c)
        do_add(10)
        do_add(11)

        for aj in range(12 - LS, 12):
            out_cps[aj].wait()
        for rr in y_rdmas:
            rr.wait_send()
        for rr in fwd_rdmas:
            rr.wait_send()

    return pl.pallas_call(
        body,
        out_shape=jax.ShapeDtypeStruct((m, n), x.dtype),
        in_specs=[pl.BlockSpec(memory_space=pltpu.MemorySpace.HBM)],
        out_specs=pl.BlockSpec(memory_space=pltpu.MemorySpace.HBM),
        scratch_shapes=[
            pltpu.VMEM((2816, n), x.dtype),
            pltpu.VMEM((2688, n), x.dtype),
            pltpu.VMEM((2688, n), x.dtype),
            pltpu.VMEM((LS, 768, n), x.dtype),
            pltpu.SemaphoreType.DMA((LS,)),
            pltpu.SemaphoreType.DMA((LS,)),
            pltpu.SemaphoreType.DMA((22,)),
            pltpu.SemaphoreType.DMA((22,)),
            pltpu.SemaphoreType.DMA((21,)),
            pltpu.SemaphoreType.DMA((21,)),
            pltpu.SemaphoreType.DMA((21,)),
            pltpu.SemaphoreType.DMA((21,)),
        ],
        compiler_params=pltpu.CompilerParams(
            collective_id=0, vmem_limit_bytes=60 * 1024 * 1024,
        ),
    )(x)
